# baseline (speedup 1.0000x reference)
import jax
import jax.numpy as jnp
import numpy as np
from jax.experimental import pallas as pl


def _matmul_body(emb_ref, att_ref, w_ref, a_ref, b_ref):
    BL = emb_ref.shape[0] * emb_ref.shape[1]
    D = emb_ref.shape[2]
    e2d = emb_ref[...].reshape(BL, D)
    w1 = w_ref[:, :D]
    w2 = w_ref[:, D:]
    a = jax.lax.dot_general(e2d, w1, (((1,), (1,)), ((), ())),
                            preferred_element_type=jnp.float32)
    bv = jax.lax.dot_general(e2d, w2, (((1,), (1,)), ((), ())),
                             preferred_element_type=jnp.float32)
    a_ref[...] = jnp.where(att_ref[...] > 0, a, jnp.float32(-jnp.inf))
    b_ref[...] = bv


def _expand_body(a_ref, b_ref, att_ref, t_ref, kidx_ref, out_ref):
    L = b_ref.shape[1]
    LNL = a_ref.shape[2]
    btiled = jax.lax.dot_general(b_ref[0], t_ref[...], (((1,), (0,)), ((), ())),
                                 preferred_element_type=jnp.float32)
    val = btiled + a_ref[0]
    jg = jax.lax.broadcasted_iota(jnp.int32, (L, LNL), 0)
    bad = (kidx_ref[0] == jg) | (att_ref[0] <= 0)
    out_ref[0] = jnp.where(bad, jnp.float32(-jnp.inf), val)


def kernel(emb_sentences, att_sentences, W):
    B, L, D = emb_sentences.shape
    NL = W.shape[0]
    LNL = L * NL
    att_f = att_sentences.astype(jnp.float32)

    a_part, b_part = pl.pallas_call(
        _matmul_body,
        in_specs=[
            pl.BlockSpec((B, L, D), lambda: (0, 0, 0)),
            pl.BlockSpec((B * L, 1), lambda: (0, 0)),
            pl.BlockSpec((NL, 2 * D), lambda: (0, 0)),
        ],
        out_specs=[
            pl.BlockSpec((B * L, NL), lambda: (0, 0)),
            pl.BlockSpec((B * L, NL), lambda: (0, 0)),
        ],
        out_shape=[
            jax.ShapeDtypeStruct((B * L, NL), jnp.float32),
            jax.ShapeDtypeStruct((B * L, NL), jnp.float32),
        ],
    )(emb_sentences, att_f.reshape(B * L, 1), W)

    a_flat = a_part.reshape(B, LNL)
    b_rows = b_part.reshape(B, L, NL)

    tile_mat = jnp.asarray(
        np.arange(LNL) % NL == np.arange(NL)[:, None], dtype=jnp.float32)
    kidx = jnp.asarray((np.arange(LNL) // NL)[None, :], dtype=jnp.int32)

    out3 = pl.pallas_call(
        _expand_body,
        grid=(B,),
        in_specs=[
            pl.BlockSpec((1, 1, LNL), lambda b: (b, 0, 0)),
            pl.BlockSpec((1, L, NL), lambda b: (b, 0, 0)),
            pl.BlockSpec((1, L, 1), lambda b: (b, 0, 0)),
            pl.BlockSpec((NL, LNL), lambda b: (0, 0)),
            pl.BlockSpec((1, 1, LNL), lambda b: (0, 0, 0)),
        ],
        out_specs=pl.BlockSpec((1, L, LNL), lambda b: (b, 0, 0)),
        out_shape=jax.ShapeDtypeStruct((B, L, LNL), jnp.float32),
    )(a_flat.reshape(B, 1, LNL), b_rows, att_f.reshape(B, L, 1),
      tile_mat, kidx.reshape(1, 1, LNL))
    return out3.reshape(B, L * L, NL)


# speedup vs baseline: 2.6407x; 2.6407x over previous
import jax
import jax.numpy as jnp
from jax.experimental import pallas as pl


def _body(emb_ref, att_ref, w_ref, out_ref):
    L, D = emb_ref.shape[1], emb_ref.shape[2]
    NL = w_ref.shape[0]
    e = emb_ref[0]
    a = jax.lax.dot_general(e, w_ref[:, :D], (((1,), (1,)), ((), ())),
                            preferred_element_type=jnp.float32)
    bv = jax.lax.dot_general(e, w_ref[:, D:], (((1,), (1,)), ((), ())),
                             preferred_element_type=jnp.float32)
    attc = att_ref[0]
    neg_inf = jnp.float32(-jnp.inf)
    a = jnp.where(attc > 0, a, neg_inf)
    bv = jnp.where(attc > 0, bv, neg_inf)
    JC = 8
    for jc in range(L // JC):
        bchunk = bv[jc * JC:(jc + 1) * JC]
        blk = a[None, :, :] + bchunk[:, None, :]
        jg = jc * JC + jax.lax.broadcasted_iota(jnp.int32, (JC, L, 1), 0)
        kg = jax.lax.broadcasted_iota(jnp.int32, (JC, L, 1), 1)
        blk = jnp.where(jg == kg, neg_inf, blk)
        out_ref[0, jc * JC * L:(jc + 1) * JC * L, :] = blk.reshape(JC * L, NL)


def kernel(emb_sentences, att_sentences, W):
    B, L, D = emb_sentences.shape
    NL = W.shape[0]
    att_col = att_sentences.astype(jnp.float32).reshape(B, L, 1)
    return pl.pallas_call(
        _body,
        grid=(B,),
        in_specs=[
            pl.BlockSpec((1, L, D), lambda b: (b, 0, 0)),
            pl.BlockSpec((1, L, 1), lambda b: (b, 0, 0)),
            pl.BlockSpec((NL, 2 * D), lambda b: (0, 0)),
        ],
        out_specs=pl.BlockSpec((1, L * L, NL), lambda b: (b, 0, 0)),
        out_shape=jax.ShapeDtypeStruct((B, L * L, NL), jnp.float32),
    )(emb_sentences, att_col, W)
